# baseline (speedup 1.0000x reference)
import functools

import jax
import jax.numpy as jnp
from jax import lax
from jax.experimental import pallas as pl
from jax.experimental.pallas import tpu as pltpu
from jax.experimental.pallas import tpu_sc as plsc

_D = 32
_G = 128
_NG = 8
_R = _G * _NG


@functools.partial(jax.jit, static_argnums=(2, 3))
def _sc_gather(xf, embeds, n_rows, n_workers):
    rows_per_worker = n_rows // n_workers
    chunks = rows_per_worker // _R
    obs = n_rows // _G
    obs_per_worker = rows_per_worker // _G

    mesh = plsc.VectorSubcoreMesh(core_axis_name="c", subcore_axis_name="s")

    @functools.partial(
        pl.kernel,
        out_type=jax.ShapeDtypeStruct((_D // 8, obs, 8, _G), jnp.float32),
        mesh=mesh,
        scratch_types=[
            pltpu.VMEM((2, _R), jnp.int32),
            pltpu.VMEM((2, _NG, _G, _D), jnp.float32),
            pltpu.SemaphoreType.DMA,
            pltpu.SemaphoreType.DMA,
            pltpu.SemaphoreType.DMA,
            pltpu.SemaphoreType.DMA,
            pltpu.SemaphoreType.DMA,
            pltpu.SemaphoreType.DMA,
        ],
        compiler_params=pltpu.CompilerParams(
            use_tc_tiling_on_sc=False, needs_layout_passes=False
        ),
    )
    def body(x_hbm, tab_hbm, out_hbm, idx_v, rows_v, i0, i1, g0, g1, o0, o1):
        wid = lax.axis_index("s") * mesh.num_cores + lax.axis_index("c")
        row_base = wid * rows_per_worker
        ob_base = wid * obs_per_worker
        isems = (i0, i1)
        gsems = (g0, g1)
        osems = (o0, o1)

        def icopy(c, b):
            row0 = row_base + c * _R
            return pltpu.make_async_copy(
                x_hbm.at[pl.ds(row0, _R)], idx_v.at[b], isems[b]
            )

        def gcopy(b, g):
            return pltpu.make_async_copy(
                tab_hbm.at[idx_v.at[b].at[pl.ds(g * _G, _G)]],
                rows_v.at[b].at[g],
                gsems[b],
            )

        def ocopy(c, b, dd):
            ob0 = ob_base + c * _NG
            return pltpu.make_async_copy(
                rows_v.at[b].at[:, :, dd],
                out_hbm.at[dd // 8, pl.ds(ob0, _NG), dd % 8],
                osems[b],
            )

        def ostart_all(c, b):
            def f(dd, carry):
                ocopy(c, b, dd).start()
                return carry
            lax.fori_loop(0, _D, f, 0)

        def owait_all(c, b):
            def f(dd, carry):
                ocopy(c, b, dd).wait()
                return carry
            lax.fori_loop(0, _D, f, 0)

        icopy(0, 0).start()
        icopy(1, 1).start()
        icopy(0, 0).wait()
        for g in range(_NG):
            gcopy(0, g).start()

        def step(c, b, first, last):
            for g in range(_NG):
                gcopy(b, g).wait()
            if not last:
                icopy(c + 1, 1 - b).wait()
            ostart_all(c, b)
            if not first:
                owait_all(c - 1, 1 - b)
            if not last:
                for g in range(_NG):
                    gcopy(1 - b, g).start()
            nxt = jnp.minimum(c + 2, chunks - 1)
            icopy(nxt, b).start()

        step(0, 0, True, False)
        step(1, 1, False, False)

        def loop(i, carry):
            step(2 * i, 0, False, False)
            step(2 * i + 1, 1, False, False)
            return carry

        lax.fori_loop(1, chunks // 2 - 1, loop, 0)
        step(chunks - 2, 0, False, False)
        step(chunks - 1, 1, False, True)

        icopy(chunks - 1, 0).wait()
        icopy(chunks - 1, 1).wait()
        owait_all(chunks - 1, 1)

    return body(xf, embeds)


def kernel(x, embeds):
    n = x.size
    xf = x.reshape(-1).astype(jnp.int32)
    op = _sc_gather(xf, embeds, n, 32)
    return op.transpose(1, 3, 0, 2).reshape(n, _D)


# speedup vs baseline: 154.4526x; 154.4526x over previous
import functools

import jax
import jax.numpy as jnp
from jax import lax
from jax.experimental import pallas as pl
from jax.experimental.pallas import tpu as pltpu
from jax.experimental.pallas import tpu_sc as plsc

_D = 32
_G = 128
_NG = 4
_R = _G * _NG


@functools.partial(jax.jit, static_argnums=(2, 3))
def _sc_gather(xf, embeds, n_rows, n_workers):
    rows_per_worker = n_rows // n_workers
    chunks = rows_per_worker // _R
    obs = n_rows // _G
    obs_per_worker = rows_per_worker // _G

    mesh = plsc.VectorSubcoreMesh(core_axis_name="c", subcore_axis_name="s")

    @functools.partial(
        pl.kernel,
        out_type=jax.ShapeDtypeStruct((_D // 8, obs, 8, _G), jnp.float32),
        mesh=mesh,
        scratch_types=[
            pltpu.VMEM((2, _R), jnp.int32),
            pltpu.VMEM((2, _R, _D), jnp.float32),
            pltpu.VMEM((2, _NG, _D, _G), jnp.float32),
            pltpu.SemaphoreType.DMA,
            pltpu.SemaphoreType.DMA,
            pltpu.SemaphoreType.DMA,
            pltpu.SemaphoreType.DMA,
            pltpu.SemaphoreType.DMA,
            pltpu.SemaphoreType.DMA,
        ],
        compiler_params=pltpu.CompilerParams(
            use_tc_tiling_on_sc=False, needs_layout_passes=False
        ),
    )
    def body(x_hbm, tab_hbm, out_hbm, idx_v, rows_v, trows_v,
             i0, i1, g0, g1, o0, o1):
        wid = lax.axis_index("s") * mesh.num_cores + lax.axis_index("c")
        row_base = wid * rows_per_worker
        ob_base = wid * obs_per_worker
        isems = (i0, i1)
        gsems = (g0, g1)
        osems = (o0, o1)

        def icopy(c, b):
            row0 = row_base + c * _R
            return pltpu.make_async_copy(
                x_hbm.at[pl.ds(row0, _R)], idx_v.at[b], isems[b]
            )

        def gcopy(b):
            return pltpu.make_async_copy(
                tab_hbm.at[idx_v.at[b]], rows_v.at[b], gsems[b]
            )

        def ocopy(c, b, g, db):
            ob = ob_base + c * _NG + g
            return pltpu.make_async_copy(
                trows_v.at[b, g, pl.ds(8 * db, 8)],
                out_hbm.at[db, ob],
                osems[b],
            )

        iota16 = lax.iota(jnp.int32, 16)
        dsels = [d0 + (iota16 + j) % 16 for d0 in (0, 16) for j in range(16)]

        def transpose_and_emit(c, b, wait_out):
            if wait_out:
                for g in range(_NG):
                    for db in range(_D // 8):
                        ocopy(c, b, g, db).wait()
            for g in range(_NG):
                def tblk(tb, carry):
                    rows16 = g * _G + tb * 16 + iota16
                    cols16 = tb * 16 + iota16
                    for dsel in dsels:
                        v = plsc.load_gather(rows_v.at[b], [rows16, dsel])
                        plsc.store_scatter(
                            trows_v.at[b, g], [dsel, cols16], v
                        )
                    return carry
                lax.fori_loop(0, _G // 16, tblk, 0)
            for g in range(_NG):
                for db in range(_D // 8):
                    ocopy(c, b, g, db).start()

        icopy(0, 0).start()
        icopy(1, 1).start()
        icopy(0, 0).wait()
        gcopy(0).start()

        def step(c, b, wait_out):
            gcopy(b).wait()

            @pl.when(c + 1 < chunks)
            def _():
                icopy(c + 1, 1 - b).wait()
                gcopy(1 - b).start()

            transpose_and_emit(c, b, wait_out)
            nxt = jnp.minimum(c + 2, chunks - 1)
            icopy(nxt, b).start()

        step(0, 0, False)
        step(1, 1, False)

        def loop(i, carry):
            step(2 * i, 0, True)
            step(2 * i + 1, 1, True)
            return carry

        lax.fori_loop(1, chunks // 2, loop, 0)

        icopy(chunks - 1, 0).wait()
        icopy(chunks - 1, 1).wait()
        for b in (0, 1):
            for g in range(_NG):
                for db in range(_D // 8):
                    ocopy(chunks - 2 + b, b, g, db).wait()

    return body(xf, embeds)


def kernel(x, embeds):
    n = x.size
    xf = x.reshape(-1).astype(jnp.int32)
    op = _sc_gather(xf, embeds, n, 32)
    return op.transpose(1, 3, 0, 2).reshape(n, _D)


# speedup vs baseline: 179.5444x; 1.1625x over previous
import functools

import jax
import jax.numpy as jnp
from jax import lax
from jax.experimental import pallas as pl
from jax.experimental.pallas import tpu as pltpu
from jax.experimental.pallas import tpu_sc as plsc

_D = 32
_G = 128
_NG = 4
_R = _G * _NG


@functools.partial(jax.jit, static_argnums=(2, 3))
def _sc_gather(xf, embeds, n_rows, n_workers):
    rows_per_worker = n_rows // n_workers
    chunks = rows_per_worker // _R
    obs = n_rows // _G
    obs_per_worker = rows_per_worker // _G

    mesh = plsc.VectorSubcoreMesh(core_axis_name="c", subcore_axis_name="s")

    @functools.partial(
        pl.kernel,
        out_type=jax.ShapeDtypeStruct((_D // 8, obs, 8, _G), jnp.float32),
        mesh=mesh,
        scratch_types=[
            pltpu.VMEM((2, _R), jnp.int32),
            pltpu.VMEM((2, _R, _D), jnp.float32),
            pltpu.VMEM((2, _NG, _D, _G), jnp.float32),
            pltpu.SemaphoreType.DMA,
            pltpu.SemaphoreType.DMA,
            pltpu.SemaphoreType.DMA,
            pltpu.SemaphoreType.DMA,
            pltpu.SemaphoreType.DMA,
            pltpu.SemaphoreType.DMA,
        ],
        compiler_params=pltpu.CompilerParams(
            use_tc_tiling_on_sc=False, needs_layout_passes=False
        ),
    )
    def body(x_hbm, tab_hbm, out_hbm, idx_v, rows_v, trows_v,
             i0, i1, g0, g1, o0, o1):
        wid = lax.axis_index("s") * mesh.num_cores + lax.axis_index("c")
        row_base = wid * rows_per_worker
        ob_base = wid * obs_per_worker
        isems = (i0, i1)
        gsems = (g0, g1)
        osems = (o0, o1)

        def icopy(c, b):
            row0 = row_base + c * _R
            return pltpu.make_async_copy(
                x_hbm.at[pl.ds(row0, _R)], idx_v.at[b], isems[b]
            )

        def gcopy(b):
            return pltpu.make_async_copy(
                tab_hbm.at[idx_v.at[b]], rows_v.at[b], gsems[b]
            )

        def ocopy(c, b, g, db):
            ob = ob_base + c * _NG + g
            return pltpu.make_async_copy(
                trows_v.at[b, g, pl.ds(8 * db, 8)],
                out_hbm.at[db, ob],
                osems[b],
            )

        iota16 = lax.iota(jnp.int32, 16)
        dsels = [d0 + (iota16 + j) % 16 for d0 in (0, 16) for j in range(16)]

        def transpose_and_emit(c, b, wait_out):
            if wait_out:
                for g in range(_NG):
                    for db in range(_D // 8):
                        ocopy(c, b, g, db).wait()
            for g in range(_NG):
                def tblk(tb, carry):
                    rows16 = g * _G + tb * 16 + iota16
                    cols16 = tb * 16 + iota16
                    for dsel in dsels:
                        v = plsc.load_gather(rows_v.at[b], [rows16, dsel])
                        plsc.store_scatter(
                            trows_v.at[b, g], [dsel, cols16], v
                        )
                    return carry
                lax.fori_loop(0, _G // 16, tblk, 0)
            for g in range(_NG):
                for db in range(_D // 8):
                    ocopy(c, b, g, db).start()

        icopy(0, 0).start()
        icopy(1, 1).start()
        icopy(0, 0).wait()
        gcopy(0).start()

        def step(c, b, wait_out):
            gcopy(b).wait()

            @pl.when(c + 1 < chunks)
            def _():
                icopy(c + 1, 1 - b).wait()
                gcopy(1 - b).start()

            transpose_and_emit(c, b, wait_out)
            nxt = jnp.minimum(c + 2, chunks - 1)
            icopy(nxt, b).start()

        step(0, 0, False)
        step(1, 1, False)

        def loop(i, carry):
            step(2 * i, 0, True)
            step(2 * i + 1, 1, True)
            return carry

        lax.fori_loop(1, chunks // 2, loop, 0)

        icopy(chunks - 1, 0).wait()
        icopy(chunks - 1, 1).wait()
        for b in (0, 1):
            for g in range(_NG):
                for db in range(_D // 8):
                    ocopy(chunks - 2 + b, b, g, db).wait()

    return body(xf, embeds)


_TS = 512


@jax.jit
def _sc_table_rowmajor(embt, tail):
    v = embt.shape[1]
    slabs = v // _TS
    vmain = slabs * _TS
    ntail = v - vmain
    n_workers = 32
    mesh = plsc.VectorSubcoreMesh(core_axis_name="c", subcore_axis_name="s")

    @functools.partial(
        pl.kernel,
        out_type=jax.ShapeDtypeStruct((v * _D // 128, 128), jnp.float32),
        mesh=mesh,
        scratch_types=[
            pltpu.VMEM((2, _D, _TS), jnp.float32),
            pltpu.VMEM((2, _TS * _D // 128, 128), jnp.float32),
            pltpu.VMEM((ntail * _D // 128, 128), jnp.float32),
            pltpu.SemaphoreType.DMA,
            pltpu.SemaphoreType.DMA,
            pltpu.SemaphoreType.DMA,
        ],
        compiler_params=pltpu.CompilerParams(
            use_tc_tiling_on_sc=True, needs_layout_passes=False
        ),
    )
    def body(embt_hbm, tail_hbm, out_hbm, slab_v, trows_v, ttail_v,
             isem, osem, tsem):
        wid = lax.axis_index("s") * mesh.num_cores + lax.axis_index("c")
        my_slabs = (slabs - 1 - wid) // n_workers + 1

        iota16 = lax.iota(jnp.int32, 16)
        dsels = [d0 + (iota16 + j) % 16 for d0 in (0, 16) for j in range(16)]

        def icopy(s, b):
            return pltpu.make_async_copy(
                embt_hbm.at[:, pl.ds(s * _TS, _TS)], slab_v.at[b], isem
            )

        def ocopy(s, b):
            l0 = s * (_TS * _D // 128)
            return pltpu.make_async_copy(
                trows_v.at[b], out_hbm.at[pl.ds(l0, _TS * _D // 128)], osem
            )

        def transpose_slab(b):
            def tblk(tb, carry):
                t16 = tb * 16 + iota16
                for dsel in dsels:
                    v16 = plsc.load_gather(slab_v.at[b], [dsel, t16])
                    flat = (tb * 16 + iota16) * _D + dsel
                    plsc.store_scatter(
                        trows_v.at[b],
                        [flat // 128, lax.rem(flat, 128)],
                        v16,
                    )
                return carry
            lax.fori_loop(0, _TS // 16, tblk, 0)

        def slab_of(i):
            return i * n_workers + wid

        @pl.when(my_slabs > 0)
        def _():
            icopy(slab_of(0), 0).start()

            def step(i, carry):
                b = lax.rem(i, 2)
                pltpu.make_async_copy(
                    embt_hbm.at[:, pl.ds(0, _TS)], slab_v.at[b], isem
                ).wait()

                @pl.when(i + 1 < my_slabs)
                def _():
                    icopy(slab_of(i + 1), 1 - b).start()

                @pl.when(i >= 2)
                def _():
                    ocopy(slab_of(i - 2), b).wait()

                transpose_slab(b)
                ocopy(slab_of(i), b).start()
                return carry

            lax.fori_loop(0, my_slabs, step, 0)
            @pl.when(my_slabs > 1)
            def _():
                ocopy(0, 0).wait()
            ocopy(0, 0).wait()

        @pl.when(wid == n_workers - 1)
        def _():
            pltpu.sync_copy(tail_hbm, ttail_v)
            pltpu.sync_copy(
                ttail_v, out_hbm.at[pl.ds(vmain * _D // 128, ntail * _D // 128)]
            )

    return body(embt, tail)


def kernel(x, embeds):
    n = x.size
    v = embeds.shape[0]
    xf = x.reshape(-1).astype(jnp.int32)
    vmain = (v // _TS) * _TS
    tail = lax.slice(embeds, (vmain, 0), (v, _D)).reshape(-1, 128)
    tab128 = _sc_table_rowmajor(embeds.T, tail)
    tab_lin = tab128.reshape(v, _D)
    op = _sc_gather(xf, tab_lin, n, 32)
    return op.transpose(1, 3, 0, 2).reshape(n, _D)


# speedup vs baseline: 181.4978x; 1.0109x over previous
import functools

import jax
import jax.numpy as jnp
from jax import lax
from jax.experimental import pallas as pl
from jax.experimental.pallas import tpu as pltpu
from jax.experimental.pallas import tpu_sc as plsc

_D = 32
_G = 128
_NG = 4
_R = _G * _NG


@functools.partial(jax.jit, static_argnums=(2, 3))
def _sc_gather(xf, embeds, n_rows, n_workers):
    rows_per_worker = n_rows // n_workers
    chunks = rows_per_worker // _R
    obs = n_rows // _G
    obs_per_worker = rows_per_worker // _G

    mesh = plsc.VectorSubcoreMesh(core_axis_name="c", subcore_axis_name="s")

    @functools.partial(
        pl.kernel,
        out_type=jax.ShapeDtypeStruct((_D // 8, obs, 8, _G), jnp.float32),
        mesh=mesh,
        scratch_types=[
            pltpu.VMEM((2, _R), jnp.int32),
            pltpu.VMEM((2, _R, _D), jnp.float32),
            pltpu.VMEM((2, _NG, _D, _G), jnp.float32),
            pltpu.SemaphoreType.DMA,
            pltpu.SemaphoreType.DMA,
            pltpu.SemaphoreType.DMA,
            pltpu.SemaphoreType.DMA,
            pltpu.SemaphoreType.DMA,
            pltpu.SemaphoreType.DMA,
        ],
        compiler_params=pltpu.CompilerParams(
            use_tc_tiling_on_sc=False, needs_layout_passes=False
        ),
    )
    def body(x_hbm, tab_hbm, out_hbm, idx_v, rows_v, trows_v,
             i0, i1, g0, g1, o0, o1):
        wid = lax.axis_index("s") * mesh.num_cores + lax.axis_index("c")
        row_base = wid * rows_per_worker
        ob_base = wid * obs_per_worker
        isems = (i0, i1)
        gsems = (g0, g1)
        osems = (o0, o1)

        def icopy(c, b):
            row0 = row_base + c * _R
            return pltpu.make_async_copy(
                x_hbm.at[pl.ds(row0, _R)], idx_v.at[b], isems[b]
            )

        def gcopy(b):
            return pltpu.make_async_copy(
                tab_hbm.at[idx_v.at[b]], rows_v.at[b], gsems[b]
            )

        def ocopy(c, b, db):
            ob0 = ob_base + c * _NG
            return pltpu.make_async_copy(
                trows_v.at[b, :, pl.ds(8 * db, 8)],
                out_hbm.at[db, pl.ds(ob0, _NG)],
                osems[b],
            )

        iota16 = lax.iota(jnp.int32, 16)
        dsels = [d0 + (iota16 + j) % 16 for d0 in (0, 16) for j in range(16)]

        def transpose_and_emit(c, b, wait_out):
            if wait_out:
                for db in range(_D // 8):
                    ocopy(c, b, db).wait()
            for g in range(_NG):
                def tblk(tb, carry):
                    rows16 = g * _G + tb * 16 + iota16
                    cols16 = tb * 16 + iota16
                    for dsel in dsels:
                        v = plsc.load_gather(rows_v.at[b], [rows16, dsel])
                        plsc.store_scatter(
                            trows_v.at[b, g], [dsel, cols16], v
                        )
                    return carry
                lax.fori_loop(0, _G // 16, tblk, 0)
            for db in range(_D // 8):
                ocopy(c, b, db).start()

        icopy(0, 0).start()
        icopy(1, 1).start()
        icopy(0, 0).wait()
        gcopy(0).start()

        def step(c, b, wait_out):
            gcopy(b).wait()

            @pl.when(c + 1 < chunks)
            def _():
                icopy(c + 1, 1 - b).wait()
                gcopy(1 - b).start()

            transpose_and_emit(c, b, wait_out)
            nxt = jnp.minimum(c + 2, chunks - 1)
            icopy(nxt, b).start()

        step(0, 0, False)
        step(1, 1, False)

        def loop(i, carry):
            step(2 * i, 0, True)
            step(2 * i + 1, 1, True)
            return carry

        lax.fori_loop(1, chunks // 2, loop, 0)

        icopy(chunks - 1, 0).wait()
        icopy(chunks - 1, 1).wait()
        for b in (0, 1):
            for db in range(_D // 8):
                ocopy(chunks - 2 + b, b, db).wait()

    return body(xf, embeds)


_TS = 512


@jax.jit
def _sc_table_rowmajor(embt, tail):
    v = embt.shape[1]
    slabs = v // _TS
    vmain = slabs * _TS
    ntail = v - vmain
    n_workers = 32
    mesh = plsc.VectorSubcoreMesh(core_axis_name="c", subcore_axis_name="s")

    @functools.partial(
        pl.kernel,
        out_type=jax.ShapeDtypeStruct((v * _D // 128, 128), jnp.float32),
        mesh=mesh,
        scratch_types=[
            pltpu.VMEM((2, _D, _TS), jnp.float32),
            pltpu.VMEM((2, _TS * _D // 128, 128), jnp.float32),
            pltpu.VMEM((ntail * _D // 128, 128), jnp.float32),
            pltpu.SemaphoreType.DMA,
            pltpu.SemaphoreType.DMA,
            pltpu.SemaphoreType.DMA,
        ],
        compiler_params=pltpu.CompilerParams(
            use_tc_tiling_on_sc=True, needs_layout_passes=False
        ),
    )
    def body(embt_hbm, tail_hbm, out_hbm, slab_v, trows_v, ttail_v,
             isem, osem, tsem):
        wid = lax.axis_index("s") * mesh.num_cores + lax.axis_index("c")
        my_slabs = (slabs - 1 - wid) // n_workers + 1

        iota16 = lax.iota(jnp.int32, 16)
        dsels = [d0 + (iota16 + j) % 16 for d0 in (0, 16) for j in range(16)]

        def icopy(s, b):
            return pltpu.make_async_copy(
                embt_hbm.at[:, pl.ds(s * _TS, _TS)], slab_v.at[b], isem
            )

        def ocopy(s, b):
            l0 = s * (_TS * _D // 128)
            return pltpu.make_async_copy(
                trows_v.at[b], out_hbm.at[pl.ds(l0, _TS * _D // 128)], osem
            )

        def transpose_slab(b):
            def tblk(tb, carry):
                t16 = tb * 16 + iota16
                for dsel in dsels:
                    v16 = plsc.load_gather(slab_v.at[b], [dsel, t16])
                    flat = (tb * 16 + iota16) * _D + dsel
                    plsc.store_scatter(
                        trows_v.at[b],
                        [flat // 128, lax.rem(flat, 128)],
                        v16,
                    )
                return carry
            lax.fori_loop(0, _TS // 16, tblk, 0)

        def slab_of(i):
            return i * n_workers + wid

        @pl.when(my_slabs > 0)
        def _():
            icopy(slab_of(0), 0).start()

            def step(i, carry):
                b = lax.rem(i, 2)
                pltpu.make_async_copy(
                    embt_hbm.at[:, pl.ds(0, _TS)], slab_v.at[b], isem
                ).wait()

                @pl.when(i + 1 < my_slabs)
                def _():
                    icopy(slab_of(i + 1), 1 - b).start()

                @pl.when(i >= 2)
                def _():
                    ocopy(slab_of(i - 2), b).wait()

                transpose_slab(b)
                ocopy(slab_of(i), b).start()
                return carry

            lax.fori_loop(0, my_slabs, step, 0)
            @pl.when(my_slabs > 1)
            def _():
                ocopy(0, 0).wait()
            ocopy(0, 0).wait()

        @pl.when(wid == n_workers - 1)
        def _():
            pltpu.sync_copy(tail_hbm, ttail_v)
            pltpu.sync_copy(
                ttail_v, out_hbm.at[pl.ds(vmain * _D // 128, ntail * _D // 128)]
            )

    return body(embt, tail)


def kernel(x, embeds):
    n = x.size
    v = embeds.shape[0]
    xf = x.reshape(-1).astype(jnp.int32)
    vmain = (v // _TS) * _TS
    tail = lax.slice(embeds, (vmain, 0), (v, _D)).reshape(-1, 128)
    tab128 = _sc_table_rowmajor(embeds.T, tail)
    tab_lin = tab128.reshape(v, _D)
    op = _sc_gather(xf, tab_lin, n, 32)
    return op.transpose(1, 3, 0, 2).reshape(n, _D)


# speedup vs baseline: 459.7337x; 2.5330x over previous
import functools

import jax
import jax.numpy as jnp
from jax import lax
from jax.experimental import pallas as pl
from jax.experimental.pallas import tpu as pltpu
from jax.experimental.pallas import tpu_sc as plsc

_D = 32
_G = 128
_NG = 4
_R = _G * _NG


@functools.partial(jax.jit, static_argnums=(2, 3))
def _sc_gather(xf, embeds, n_rows, n_workers):
    rows_per_worker = n_rows // n_workers
    chunks = rows_per_worker // _R
    obs = n_rows // _G
    obs_per_worker = rows_per_worker // _G

    mesh = plsc.VectorSubcoreMesh(core_axis_name="c", subcore_axis_name="s")

    @functools.partial(
        pl.kernel,
        out_type=jax.ShapeDtypeStruct((_D // 8, obs, 8, _G), jnp.float32),
        mesh=mesh,
        scratch_types=[
            pltpu.VMEM((2, _R), jnp.int32),
            pltpu.VMEM((2, _R, _D), jnp.float32),
            pltpu.VMEM((2, _NG, _D, _G), jnp.float32),
            pltpu.SemaphoreType.DMA,
            pltpu.SemaphoreType.DMA,
            pltpu.SemaphoreType.DMA,
            pltpu.SemaphoreType.DMA,
            pltpu.SemaphoreType.DMA,
            pltpu.SemaphoreType.DMA,
        ],
        compiler_params=pltpu.CompilerParams(
            use_tc_tiling_on_sc=False, needs_layout_passes=False
        ),
    )
    def body(x_hbm, tab_hbm, out_hbm, idx_v, rows_v, trows_v,
             i0, i1, g0, g1, o0, o1):
        wid = lax.axis_index("s") * mesh.num_cores + lax.axis_index("c")
        row_base = wid * rows_per_worker
        ob_base = wid * obs_per_worker
        isems = (i0, i1)
        gsems = (g0, g1)
        osems = (o0, o1)

        def icopy(c, b):
            row0 = row_base + c * _R
            return pltpu.make_async_copy(
                x_hbm.at[pl.ds(row0, _R)], idx_v.at[b], isems[b]
            )

        def gcopy(b):
            return pltpu.make_async_copy(
                tab_hbm.at[idx_v.at[b]], rows_v.at[b], gsems[b]
            )

        def ocopy(c, b, db):
            ob0 = ob_base + c * _NG
            return pltpu.make_async_copy(
                trows_v.at[b, :, pl.ds(8 * db, 8)],
                out_hbm.at[db, pl.ds(ob0, _NG)],
                osems[b],
            )

        iota16 = lax.iota(jnp.int32, 16)
        dsels = [d0 + (iota16 + j) % 16 for d0 in (0, 16) for j in range(16)]

        def transpose_and_emit(c, b, wait_out):
            if wait_out:
                for db in range(_D // 8):
                    ocopy(c, b, db).wait()
            for g in range(_NG):
                @functools.partial(
                    plsc.parallel_loop, 0, _G // 16, unroll=2
                )
                def tblk(tb):
                    rows16 = g * _G + tb * 16 + iota16
                    cols16 = tb * 16 + iota16
                    for dsel in dsels:
                        v = plsc.load_gather(rows_v.at[b], [rows16, dsel])
                        plsc.store_scatter(
                            trows_v.at[b, g], [dsel, cols16], v
                        )
            for db in range(_D // 8):
                ocopy(c, b, db).start()

        icopy(0, 0).start()
        icopy(1, 1).start()
        icopy(0, 0).wait()
        gcopy(0).start()

        def step(c, b, wait_out):
            gcopy(b).wait()

            @pl.when(c + 1 < chunks)
            def _():
                icopy(c + 1, 1 - b).wait()
                gcopy(1 - b).start()

            transpose_and_emit(c, b, wait_out)
            nxt = jnp.minimum(c + 2, chunks - 1)
            icopy(nxt, b).start()

        step(0, 0, False)
        step(1, 1, False)

        def loop(i, carry):
            step(2 * i, 0, True)
            step(2 * i + 1, 1, True)
            return carry

        lax.fori_loop(1, chunks // 2, loop, 0)

        icopy(chunks - 1, 0).wait()
        icopy(chunks - 1, 1).wait()
        for b in (0, 1):
            for db in range(_D // 8):
                ocopy(chunks - 2 + b, b, db).wait()

    return body(xf, embeds)


_TS = 512


@jax.jit
def _sc_table_rowmajor(embt, tail):
    v = embt.shape[1]
    slabs = v // _TS
    vmain = slabs * _TS
    ntail = v - vmain
    n_workers = 32
    mesh = plsc.VectorSubcoreMesh(core_axis_name="c", subcore_axis_name="s")

    @functools.partial(
        pl.kernel,
        out_type=jax.ShapeDtypeStruct((v * _D // 128, 128), jnp.float32),
        mesh=mesh,
        scratch_types=[
            pltpu.VMEM((2, _D, _TS), jnp.float32),
            pltpu.VMEM((2, _TS * _D // 128, 128), jnp.float32),
            pltpu.VMEM((ntail * _D // 128, 128), jnp.float32),
            pltpu.SemaphoreType.DMA,
            pltpu.SemaphoreType.DMA,
            pltpu.SemaphoreType.DMA,
        ],
        compiler_params=pltpu.CompilerParams(
            use_tc_tiling_on_sc=True, needs_layout_passes=False
        ),
    )
    def body(embt_hbm, tail_hbm, out_hbm, slab_v, trows_v, ttail_v,
             isem, osem, tsem):
        wid = lax.axis_index("s") * mesh.num_cores + lax.axis_index("c")
        my_slabs = (slabs - 1 - wid) // n_workers + 1

        iota16 = lax.iota(jnp.int32, 16)
        dsels = [d0 + (iota16 + j) % 16 for d0 in (0, 16) for j in range(16)]

        def icopy(s, b):
            return pltpu.make_async_copy(
                embt_hbm.at[:, pl.ds(s * _TS, _TS)], slab_v.at[b], isem
            )

        def ocopy(s, b):
            l0 = s * (_TS * _D // 128)
            return pltpu.make_async_copy(
                trows_v.at[b], out_hbm.at[pl.ds(l0, _TS * _D // 128)], osem
            )

        rowcs = [(32 * iota16 + dsel) // 128 for dsel in dsels]
        colcs = [lax.rem(32 * iota16 + dsel, 128) for dsel in dsels]

        def transpose_slab(b):
            @functools.partial(
                plsc.parallel_loop, 0, _TS // 16, unroll=2
            )
            def tblk(tb):
                t16 = tb * 16 + iota16
                for dsel, rowc, colc in zip(dsels, rowcs, colcs):
                    v16 = plsc.load_gather(slab_v.at[b], [dsel, t16])
                    plsc.store_scatter(
                        trows_v.at[b], [4 * tb + rowc, colc], v16
                    )

        def slab_of(i):
            return i * n_workers + wid

        @pl.when(my_slabs > 0)
        def _():
            icopy(slab_of(0), 0).start()

            def step(i, carry):
                b = lax.rem(i, 2)
                pltpu.make_async_copy(
                    embt_hbm.at[:, pl.ds(0, _TS)], slab_v.at[b], isem
                ).wait()

                @pl.when(i + 1 < my_slabs)
                def _():
                    icopy(slab_of(i + 1), 1 - b).start()

                @pl.when(i >= 2)
                def _():
                    ocopy(slab_of(i - 2), b).wait()

                transpose_slab(b)
                ocopy(slab_of(i), b).start()
                return carry

            lax.fori_loop(0, my_slabs, step, 0)
            @pl.when(my_slabs > 1)
            def _():
                ocopy(0, 0).wait()
            ocopy(0, 0).wait()

        @pl.when(wid == n_workers - 1)
        def _():
            pltpu.sync_copy(tail_hbm, ttail_v)
            pltpu.sync_copy(
                ttail_v, out_hbm.at[pl.ds(vmain * _D // 128, ntail * _D // 128)]
            )

    return body(embt, tail)


def kernel(x, embeds):
    n = x.size
    v = embeds.shape[0]
    xf = x.reshape(-1).astype(jnp.int32)
    vmain = (v // _TS) * _TS
    tail = lax.slice(embeds, (vmain, 0), (v, _D)).reshape(-1, 128)
    tab128 = _sc_table_rowmajor(embeds.T, tail)
    tab_lin = tab128.reshape(v, _D)
    op = _sc_gather(xf, tab_lin, n, 32)
    return op.transpose(1, 3, 0, 2).reshape(n, _D)
